# Initial kernel scaffold; baseline (speedup 1.0000x reference)
import jax
import jax.numpy as jnp
from jax.experimental import pallas as pl


def kernel(x_paper, edge_index_writes, edge_index_cites, lin_paper_W, lin_paper_b, emb_author, c0_root_paper_W, c0_root_paper_b, c0_root_author_W, c0_root_author_b, c0_rel_writes_W, c0_rel_cites_W, c1_root_paper_W, c1_root_paper_b, c1_rel_writes_W, c1_rel_cites_W):
    raise NotImplementedError("write your pallas kernel here")


# speedup vs baseline: 8.8818x; 8.8818x over previous
import jax
import jax.numpy as jnp
from jax import lax
from jax.experimental import pallas as pl
from jax.experimental.pallas import tpu as pltpu
from jax.experimental.pallas import tpu_sc as plsc

N = 10000
D = 128
E = 320000
DOUT = 349
NC = 2
NS = 16
CHUNK = 100
NCHUNK = E // NS // CHUNK
G = 8
NG = NCHUNK // G
RPT = 624
REM = N - NS * RPT
ZROWS = 96
BLK = 1000


def _zero_rows(src_buf, dst_ref, s):
    for t in range(RPT // ZROWS):
        pltpu.sync_copy(src_buf.at[pl.ds(0, ZROWS)],
                        dst_ref.at[pl.ds(s * RPT + t * ZROWS, ZROWS)])
    tail = RPT - (RPT // ZROWS) * ZROWS
    if tail:
        pltpu.sync_copy(src_buf.at[pl.ds(0, tail)],
                        dst_ref.at[pl.ds(s * RPT + RPT - tail, tail)])

    @pl.when(s == NS - 1)
    def _():
        pltpu.sync_copy(src_buf.at[pl.ds(0, REM)],
                        dst_ref.at[pl.ds(NS * RPT, REM)])


def _seg_pipeline(s, table_r, src_r, dst_r, osum_r,
                  idx_s, idx_d, buf, sem0, sem1, semi0, semi1, acc):
    def zinit(k, carry):
        buf[0, k // 8, pl.ds((k % 8) * 16, 16)] = jnp.zeros((16,), jnp.float32)
        return carry
    lax.fori_loop(0, ZROWS * (D // 16), zinit, 0)
    _zero_rows(buf.at[0], acc, s)
    plsc.subcore_barrier()

    def idx_group_copies(g, slot):
        hb = pl.ds(g * G, G)
        return (pltpu.make_async_copy(src_r.at[s, hb], idx_s.at[slot], semi0),
                pltpu.make_async_copy(dst_r.at[s, hb], idx_d.at[slot], semi1))

    for cp in idx_group_copies(0, 0):
        cp.start()
        cp.wait()
    for cp in idx_group_copies(1, 1):
        cp.start()
    pltpu.async_copy(table_r.at[idx_s.at[0, 0]], buf.at[0], sem0)

    sems = (sem0, sem1)

    def group_body(g, st, last):
        for b in range(G):
            bb = b % 2
            if b == G - 1 and not last:
                for cp in idx_group_copies(g + 1, 1 - st):
                    cp.wait()
                pltpu.async_copy(table_r.at[idx_s.at[1 - st, 0]],
                                 buf.at[1 - bb], sems[1 - bb])
            elif b < G - 1:
                pltpu.async_copy(table_r.at[idx_s.at[st, b + 1]],
                                 buf.at[1 - bb], sems[1 - bb])
            pltpu.make_async_copy(table_r.at[idx_s.at[st, b]],
                                  buf.at[bb], sems[bb]).wait()
            pltpu.sync_copy(buf.at[bb], acc.at[idx_d.at[st, b]], add=True)
        if not last:
            @pl.when(g + 2 < NG)
            def _():
                for cp in idx_group_copies(g + 2, st):
                    cp.start()

    def pair(p, carry):
        group_body(p * 2, 0, False)
        group_body(p * 2 + 1, 1, False)
        return carry

    lax.fori_loop(0, (NG - 1) // 2, pair, 0)
    group_body(NG - 1, (NG - 1) % 2, True)

    plsc.subcore_barrier()
    pltpu.sync_copy(acc.at[pl.ds(s * RPT, RPT)], osum_r.at[pl.ds(s * RPT, RPT)])

    @pl.when(s == NS - 1)
    def _():
        pltpu.sync_copy(acc.at[pl.ds(NS * RPT, REM)],
                        osum_r.at[pl.ds(NS * RPT, REM)])


def _make_sc_kernel():
    mesh = plsc.VectorSubcoreMesh(core_axis_name="c", subcore_axis_name="s",
                                  num_cores=NC, num_subcores=NS)
    out_type = [jax.ShapeDtypeStruct((N, D), jnp.float32),
                jax.ShapeDtypeStruct((N, D), jnp.float32)]
    scratch = [
        pltpu.VMEM((2, G, CHUNK), jnp.int32),
        pltpu.VMEM((2, G, CHUNK), jnp.int32),
        pltpu.VMEM((2, CHUNK, D), jnp.float32),
        pltpu.SemaphoreType.DMA,
        pltpu.SemaphoreType.DMA,
        pltpu.SemaphoreType.DMA,
        pltpu.SemaphoreType.DMA,
        pltpu.VMEM_SHARED((N, D), jnp.float32),
    ]

    def body(tab_w, tab_c, src_w, dst_w, src_c, dst_c, osum_w, osum_c,
             idx_s, idx_d, buf, sem0, sem1, semi0, semi1, acc):
        c = lax.axis_index("c")
        s = lax.axis_index("s")
        args = (idx_s, idx_d, buf, sem0, sem1, semi0, semi1, acc)

        @pl.when(c == 0)
        def _():
            _seg_pipeline(s, tab_w, src_w, dst_w, osum_w, *args)

        @pl.when(c == 1)
        def _():
            _seg_pipeline(s, tab_c, src_c, dst_c, osum_c, *args)

    return pl.kernel(body, out_type=out_type, mesh=mesh, scratch_types=scratch)


def _make_counts_kernel():
    mesh = plsc.VectorSubcoreMesh(core_axis_name="c", subcore_axis_name="s",
                                  num_cores=NC, num_subcores=NS)
    out_type = [jax.ShapeDtypeStruct((N, D), jnp.float32),
                jax.ShapeDtypeStruct((N, D), jnp.float32)]
    scratch = [
        pltpu.VMEM((2, G, CHUNK), jnp.int32),
        pltpu.VMEM((CHUNK, D), jnp.float32),
        pltpu.VMEM((ZROWS, D), jnp.float32),
        pltpu.SemaphoreType.DMA,
        pltpu.VMEM_SHARED((N, D), jnp.float32),
    ]

    def count_pipeline(s, dst_r, ocnt_r, idx_d, cbuf, zbuf, semi, cnt):
        _zero_rows(zbuf, cnt, s)
        plsc.subcore_barrier()

        def idx_copy(g, slot):
            return pltpu.make_async_copy(dst_r.at[s, pl.ds(g * G, G)],
                                         idx_d.at[slot], semi)

        cp = idx_copy(0, 0)
        cp.start()
        cp.wait()
        idx_copy(1, 1).start()

        def group_body(g, st, first, last):
            if not first:
                idx_copy(g, st).wait()
            for b in range(G):
                pltpu.sync_copy(cbuf, cnt.at[idx_d.at[st, b]], add=True)
            if not last:
                @pl.when(g + 2 < NG)
                def _():
                    idx_copy(g + 2, st).start()

        group_body(0, 0, True, False)
        group_body(1, 1, False, False)

        def pair(p, carry):
            g = p * 2 + 2
            group_body(g, 0, False, False)
            group_body(g + 1, 1, False, False)
            return carry

        lax.fori_loop(0, (NG - 3) // 2, pair, 0)
        group_body(NG - 1, (NG - 1) % 2, False, True)

        plsc.subcore_barrier()
        pltpu.sync_copy(cnt.at[pl.ds(s * RPT, RPT)],
                        ocnt_r.at[pl.ds(s * RPT, RPT)])

        @pl.when(s == NS - 1)
        def _():
            pltpu.sync_copy(cnt.at[pl.ds(NS * RPT, REM)],
                            ocnt_r.at[pl.ds(NS * RPT, REM)])

    def body(dst_w, dst_c, ocnt_w, ocnt_c, idx_d, cbuf, zbuf, semi, cnt):
        c = lax.axis_index("c")
        s = lax.axis_index("s")
        ones16 = jnp.full((16,), 1.0, jnp.float32)
        zeros16 = jnp.zeros((16,), jnp.float32)

        def cinit(k, carry):
            cbuf[k // 8, pl.ds((k % 8) * 16, 16)] = ones16
            return carry
        lax.fori_loop(0, CHUNK * (D // 16), cinit, 0)

        def czinit(k, carry):
            zbuf[k // 8, pl.ds((k % 8) * 16, 16)] = zeros16
            return carry
        lax.fori_loop(0, ZROWS * (D // 16), czinit, 0)

        @pl.when(c == 0)
        def _():
            count_pipeline(s, dst_w, ocnt_w, idx_d, cbuf, zbuf, semi, cnt)

        @pl.when(c == 1)
        def _():
            count_pipeline(s, dst_c, ocnt_c, idx_d, cbuf, zbuf, semi, cnt)

    return pl.kernel(body, out_type=out_type, mesh=mesh, scratch_types=scratch)


def _rowspec(cols):
    return pl.BlockSpec((BLK, cols), lambda i: (i, 0))


def _fullspec(rows, cols):
    return pl.BlockSpec((rows, cols), lambda i: (0, 0))


def _dot(a, b):
    return jnp.dot(a, b, preferred_element_type=jnp.float32)


def _tc_a_body(x_ref, ha_ref, linW, linb, rpW, rpb, raW, rab,
               hp_ref, rootp_ref, ha2_ref):
    hp = _dot(x_ref[...], linW[...]) + linb[...]
    hp_ref[...] = hp
    rootp_ref[...] = _dot(hp, rpW[...]) + rpb[...]
    ha2_ref[...] = jnp.maximum(_dot(ha_ref[...], raW[...]) + rab[...], 0.0)


def _recip(cnt_ref):
    return 1.0 / jnp.maximum(cnt_ref[...][:, :1], 1.0)


def _tc_b_body(rootp, sw, sc_, cw, cc, Ww, Wc, W1, b1, hp2_ref, out1_ref):
    aggw = sw[...] * _recip(cw)
    aggc = sc_[...] * _recip(cc)
    hp2 = jnp.maximum(rootp[...] + _dot(aggw, Ww[...]) + _dot(aggc, Wc[...]),
                      0.0)
    hp2_ref[...] = hp2
    out1_ref[...] = _dot(hp2, W1[...]) + b1[...]


def _tc_c_body(out1, sw, sc_, cw, cc, Ww, Wc, o_ref):
    logits = (out1[...] + _dot(sw[...] * _recip(cw), Ww[...])
              + _dot(sc_[...] * _recip(cc), Wc[...]))
    m = jnp.max(logits, axis=-1, keepdims=True)
    ex = jnp.exp(logits - m)
    lse = jnp.log(jnp.sum(ex, axis=-1, keepdims=True))
    o_ref[...] = logits - m - lse


@jax.jit
def _run(x_paper, edge_index_writes, edge_index_cites, lin_paper_W,
         lin_paper_b, emb_author, c0_root_paper_W, c0_root_paper_b,
         c0_root_author_W, c0_root_author_b, c0_rel_writes_W, c0_rel_cites_W,
         c1_root_paper_W, c1_root_paper_b, c1_rel_writes_W, c1_rel_cites_W):
    src_w = edge_index_writes[0].reshape(NS, NCHUNK, CHUNK)
    dst_w = edge_index_writes[1].reshape(NS, NCHUNK, CHUNK)
    src_c = edge_index_cites[0].reshape(NS, NCHUNK, CHUNK)
    dst_c = edge_index_cites[1].reshape(NS, NCHUNK, CHUNK)
    grid = (N // BLK,)

    cnt_w, cnt_c = _make_counts_kernel()(dst_w, dst_c)

    tc_a = pl.pallas_call(
        _tc_a_body,
        grid=grid,
        in_specs=[_rowspec(D), _rowspec(D), _fullspec(D, D), _fullspec(1, D),
                  _fullspec(D, D), _fullspec(1, D), _fullspec(D, D),
                  _fullspec(1, D)],
        out_specs=[_rowspec(D), _rowspec(D), _rowspec(D)],
        out_shape=[jax.ShapeDtypeStruct((N, D), jnp.float32)] * 3,
    )
    hp, rootp, ha2 = tc_a(x_paper, emb_author, lin_paper_W,
                          lin_paper_b.reshape(1, D), c0_root_paper_W,
                          c0_root_paper_b.reshape(1, D), c0_root_author_W,
                          c0_root_author_b.reshape(1, D))

    sum_w, sum_c = _make_sc_kernel()(emb_author, hp, src_w, dst_w,
                                     src_c, dst_c)

    tc_b = pl.pallas_call(
        _tc_b_body,
        grid=grid,
        in_specs=[_rowspec(D), _rowspec(D), _rowspec(D), _rowspec(D),
                  _rowspec(D), _fullspec(D, D), _fullspec(D, D),
                  _fullspec(D, DOUT), _fullspec(1, DOUT)],
        out_specs=[_rowspec(D), _rowspec(DOUT)],
        out_shape=[jax.ShapeDtypeStruct((N, D), jnp.float32),
                   jax.ShapeDtypeStruct((N, DOUT), jnp.float32)],
    )
    hp2, out1 = tc_b(rootp, sum_w, sum_c, cnt_w, cnt_c, c0_rel_writes_W,
                     c0_rel_cites_W, c1_root_paper_W,
                     c1_root_paper_b.reshape(1, DOUT))

    sum_w2, sum_c2 = _make_sc_kernel()(ha2, hp2, src_w, dst_w, src_c, dst_c)

    tc_c = pl.pallas_call(
        _tc_c_body,
        grid=grid,
        in_specs=[_rowspec(DOUT), _rowspec(D), _rowspec(D), _rowspec(D),
                  _rowspec(D), _fullspec(D, DOUT), _fullspec(D, DOUT)],
        out_specs=_rowspec(DOUT),
        out_shape=jax.ShapeDtypeStruct((N, DOUT), jnp.float32),
    )
    return tc_c(out1, sum_w2, sum_c2, cnt_w, cnt_c, c1_rel_writes_W,
                c1_rel_cites_W)


def kernel(*args):
    return _run(*args)
